# Initial kernel scaffold; baseline (speedup 1.0000x reference)
import jax
import jax.numpy as jnp
from jax.experimental import pallas as pl


def kernel(old_data, x_base, x_centroid, edge_index_b2b, edge_index_b2c, edge_index_c2c, edge_index_c2b, edge_weight_b2b, edge_weight_b2c, edge_weight_c2c, edge_weight_c2b, batch_base, batch_centroid, has_edge_attr, params):
    raise NotImplementedError("write your pallas kernel here")


# speedup vs baseline: 4.2642x; 4.2642x over previous
import functools

import jax
import jax.numpy as jnp
from jax import lax
from jax.experimental import pallas as pl
from jax.experimental.pallas import tpu as pltpu
from jax.experimental.pallas import tpu_sc as plsc

F32 = jnp.float32
HID = 128
NB = 10000
NC = 1000
NUM_LAYERS = 2

K_ED = 40


def _prep_body(Wsrc, bsrc, Wdst, bdst, Wmlp, bmlp, eps,
               Wb0, bb0, Wc0, bc0, Wb1, bb1, Wc1, bc1):
    outs = {0: (Wb0, bb0, Wc0, bc0), 1: (Wb1, bb1, Wc1, bc1)}
    for l in range(NUM_LAYERS):
        Wb, bb, Wc, bc = outs[l]
        i_bb, i_bc, i_cc, i_cb = l * 4 + 0, l * 4 + 1, l * 4 + 2, l * 4 + 3

        def wm(i, j):
            return jnp.dot(Wsrc[i][...] if j is None else j,
                           Wmlp[i][...], preferred_element_type=F32)

        Wb[:, 0:HID] = wm(i_bb, None)
        Wb[:, HID:2 * HID] = wm(i_bc, None)
        Wc[:, 0:HID] = wm(i_cb, None)
        Wc[:, HID:2 * HID] = wm(i_cc, None)
        bb[:, 0:HID] = wm(i_bb, bsrc[i_bb][...].reshape(1, HID))
        bb[:, HID:2 * HID] = wm(i_bc, bsrc[i_bc][...].reshape(1, HID))
        bc[:, 0:HID] = wm(i_cb, bsrc[i_cb][...].reshape(1, HID))
        bc[:, HID:2 * HID] = wm(i_cc, bsrc[i_cc][...].reshape(1, HID))
        Wb[:, 2 * HID:] = (1.0 + eps[0, i_bb]) * wm(i_bb, Wdst[i_bb][...]) \
                        + (1.0 + eps[0, i_cb]) * wm(i_cb, Wdst[i_cb][...])
        Wc[:, 2 * HID:] = (1.0 + eps[0, i_bc]) * wm(i_bc, Wdst[i_bc][...]) \
                        + (1.0 + eps[0, i_cc]) * wm(i_cc, Wdst[i_cc][...])
        bb[:, 2 * HID:] = wm(i_bb, bdst[i_bb][...].reshape(1, HID)) + bmlp[i_bb][...].reshape(1, HID) \
                        + wm(i_cb, bdst[i_cb][...].reshape(1, HID)) + bmlp[i_cb][...].reshape(1, HID)
        bc[:, 2 * HID:] = wm(i_bc, bdst[i_bc][...].reshape(1, HID)) + bmlp[i_bc][...].reshape(1, HID) \
                        + wm(i_cc, bdst[i_cc][...].reshape(1, HID)) + bmlp[i_cc][...].reshape(1, HID)


def _prep_call(Wsrc, bsrc, Wdst, bdst, Wmlp, bmlp, eps):
    w = jax.ShapeDtypeStruct((HID, 3 * HID), F32)
    b = jax.ShapeDtypeStruct((1, 3 * HID), F32)
    return pl.pallas_call(
        _prep_body,
        out_shape=(w, b, w, b, w, b, w, b),
    )(Wsrc, bsrc, Wdst, bdst, Wmlp, bmlp, eps)


def _tc1_body(old, xc, Wa, ba, Wb, bbias, Wc, cbias,
              xb_o, hbb_o, hbc_o, sb_o, hcb_o, hcc_o, sc_o):
    xb = jnp.dot(old[...], Wa[...], preferred_element_type=F32) + ba[...]
    xb_o[...] = xb
    yb = jnp.dot(xb, Wb[...], preferred_element_type=F32) + bbias[...]
    hbb_o[...] = yb[:, 0:HID]
    hbc_o[...] = yb[:, HID:2 * HID]
    sb_o[...] = yb[:, 2 * HID:]
    yc = jnp.dot(xc[...], Wc[...], preferred_element_type=F32) + cbias[...]
    hcb_o[...] = yc[:, 0:HID]
    hcc_o[...] = yc[:, HID:2 * HID]
    sc_o[...] = yc[:, 2 * HID:]


def _tc1_call(old, xc, Wa, ba, Wb, bbias, Wc, cbias):
    b = jax.ShapeDtypeStruct((NB, HID), F32)
    c = jax.ShapeDtypeStruct((NC, HID), F32)
    return pl.pallas_call(
        _tc1_body,
        out_shape=(b, b, b, b, c, c, c),
    )(old, xc, Wa, ba, Wb, bbias, Wc, cbias)


def _tc2_body(xb, ab0, ab1, sb, xc, ac0, ac1, sc, Wb, bbias, Wc, cbias,
              xb_o, hbb_o, hbc_o, sb_o, xc_o, hcb_o, hcc_o, sc_o):
    xb1 = xb[...] + jax.nn.gelu(ab0[...] + ab1[...] + sb[...])
    xb_o[...] = xb1
    yb = jnp.dot(xb1, Wb[...], preferred_element_type=F32) + bbias[...]
    hbb_o[...] = yb[:, 0:HID]
    hbc_o[...] = yb[:, HID:2 * HID]
    sb_o[...] = yb[:, 2 * HID:]
    xc1 = xc[...] + jax.nn.gelu(ac0[...] + ac1[...] + sc[...])
    xc_o[...] = xc1
    yc = jnp.dot(xc1, Wc[...], preferred_element_type=F32) + cbias[...]
    hcb_o[...] = yc[:, 0:HID]
    hcc_o[...] = yc[:, HID:2 * HID]
    sc_o[...] = yc[:, 2 * HID:]


def _tc2_call(xb, ab0, ab1, sb, xc, ac0, ac1, sc, Wb, bbias, Wc, cbias):
    b = jax.ShapeDtypeStruct((NB, HID), F32)
    c = jax.ShapeDtypeStruct((NC, HID), F32)
    return pl.pallas_call(
        _tc2_body,
        out_shape=(b, b, b, b, c, c, c, c),
    )(xb, ab0, ab1, sb, xc, ac0, ac1, sc, Wb, bbias, Wc, cbias)


def _tc3_body(xb, ab0, ab1, sb, xc, ac0, ac1, sc, xb_o, xc_o):
    xb_o[...] = xb[...] + jax.nn.gelu(ab0[...] + ab1[...] + sb[...])
    xc_o[...] = xc[...] + jax.nn.gelu(ac0[...] + ac1[...] + sc[...])


def _tc3_call(xb, ab0, ab1, sb, xc, ac0, ac1, sc):
    return pl.pallas_call(
        _tc3_body,
        out_shape=(jax.ShapeDtypeStruct((NB, HID), F32),
                   jax.ShapeDtypeStruct((NC, HID), F32)),
    )(xb, ab0, ab1, sb, xc, ac0, ac1, sc)


def _sc_body(h_bb, h_cb, h_bc, h_cc,
             s_bb, d_bb, w_bb, s_cb, d_cb, w_cb,
             s_bc, d_bc, w_bc, s_cc, d_cc, w_cc,
             zeros,
             aggb_o, aggc_o,
             aggb_sh, aggc_sh,
             sidx, didx, wbuf, rows, sem):
    c = lax.axis_index("c")
    s = lax.axis_index("s")

    @pl.when(s < 10)
    def _():
        pltpu.sync_copy(zeros.at[:], aggb_sh.at[pl.ds(s * 1000, 1000)])

    @pl.when(s < 5)
    def _():
        pltpu.sync_copy(zeros.at[pl.ds(0, 200)], aggc_sh.at[pl.ds(s * 200, 200)])

    plsc.subcore_barrier()

    def do_rel(htab, src3, dst3, ew3, agg_sh, tiles):
        K = K_ED
        iters = src3.shape[1]
        wid = c * tiles + s
        pltpu.sync_copy(src3.at[wid], sidx.at[pl.ds(0, iters)])
        pltpu.sync_copy(dst3.at[wid], didx.at[pl.ds(0, iters)])
        pltpu.sync_copy(ew3.at[wid], wbuf.at[pl.ds(0, iters)])

        def _scale_group(g, e0, w0, lanes):
            wv = wbuf[g, pl.ds(w0, 16)]
            for u in lanes:
                e = e0 + u
                w = wv[u]
                for j in range(8):
                    sl = pl.ds(j * 16, 16)
                    rows[e, sl] = rows[e, sl] * w

        def chunk(g, carry):
            pltpu.async_copy(htab.at[sidx.at[g]], rows, sem).wait()

            def scale16(t, c2):
                _scale_group(g, t * 16, t * 16, range(16))
                return c2

            lax.fori_loop(0, K // 16, scale16, 0)
            tail = K % 16
            if tail:
                _scale_group(g, K - 16, K - 16, range(16 - tail, 16))
            pltpu.sync_copy(rows, agg_sh.at[didx.at[g]], add=True)
            return carry

        lax.fori_loop(0, iters, chunk, 0)

    do_rel(h_bb, s_bb, d_bb, w_bb, aggb_sh, 16)
    do_rel(h_cb, s_cb, d_cb, w_cb, aggb_sh, 16)
    do_rel(h_bc, s_bc, d_bc, w_bc, aggc_sh, 16)

    @pl.when(s < 8)
    def _():
        do_rel(h_cc, s_cc, d_cc, w_cc, aggc_sh, 8)

    plsc.subcore_barrier()

    @pl.when(s < 10)
    def _():
        pltpu.sync_copy(aggb_sh.at[pl.ds(s * 1000, 1000)],
                        aggb_o.at[c, pl.ds(s * 1000, 1000)])

    @pl.when(s < 5)
    def _():
        pltpu.sync_copy(aggc_sh.at[pl.ds(s * 200, 200)],
                        aggc_o.at[c, pl.ds(s * 200, 200)])


@functools.cache
def _get_sc_call():
  return functools.partial(
    pl.kernel,
    mesh=plsc.VectorSubcoreMesh(core_axis_name="c", subcore_axis_name="s",
                                num_cores=2, num_subcores=16),
    compiler_params=pltpu.CompilerParams(use_tc_tiling_on_sc=False),
    out_type=(jax.ShapeDtypeStruct((2, NB, HID), F32),
              jax.ShapeDtypeStruct((2, NC, HID), F32)),
    scratch_types=[
        pltpu.VMEM_SHARED((NB, HID), F32),
        pltpu.VMEM_SHARED((NC, HID), F32),
        pltpu.VMEM((250, K_ED), jnp.int32),
        pltpu.VMEM((250, K_ED), jnp.int32),
        pltpu.VMEM((250, K_ED), F32),
        pltpu.VMEM((K_ED, HID), F32),
        pltpu.SemaphoreType.DMA,
    ],
  )(_sc_body)


def kernel(old_data, x_base, x_centroid, edge_index_b2b, edge_index_b2c,
           edge_index_c2c, edge_index_c2b, edge_weight_b2b, edge_weight_b2c,
           edge_weight_c2c, edge_weight_c2b, batch_base, batch_centroid,
           has_edge_attr, params):
    p = params
    ets = [(l, et) for l in range(NUM_LAYERS) for et in ("bb", "bc", "cc", "cb")]
    Wsrc = jnp.stack([p[f"{l}_{et}"]["W_src"] for l, et in ets])
    bsrc = jnp.stack([p[f"{l}_{et}"]["b_src"] for l, et in ets])
    Wdst = jnp.stack([p[f"{l}_{et}"]["W_dst"] for l, et in ets])
    bdst = jnp.stack([p[f"{l}_{et}"]["b_dst"] for l, et in ets])
    Wmlp = jnp.stack([p[f"{l}_{et}"]["W_mlp"] for l, et in ets])
    bmlp = jnp.stack([p[f"{l}_{et}"]["b_mlp"] for l, et in ets])
    eps = jnp.stack([p[f"{l}_{et}"]["eps"] for l, et in ets]).reshape(1, 8)

    Wb0, bb0, Wc0, bc0, Wb1, bb1, Wc1, bc1 = _prep_call(
        Wsrc, bsrc, Wdst, bdst, Wmlp, bmlp, eps)

    s_bb = edge_index_b2b[0].reshape(32, -1, K_ED)
    d_bb = edge_index_b2b[1].reshape(32, -1, K_ED)
    w_bb = edge_weight_b2b.reshape(32, -1, K_ED)
    s_cb = edge_index_c2b[0].reshape(32, -1, K_ED)
    d_cb = edge_index_c2b[1].reshape(32, -1, K_ED)
    w_cb = edge_weight_c2b.reshape(32, -1, K_ED)
    s_bc = edge_index_b2c[0].reshape(32, -1, K_ED)
    d_bc = edge_index_b2c[1].reshape(32, -1, K_ED)
    w_bc = edge_weight_b2c.reshape(32, -1, K_ED)
    s_cc = edge_index_c2c[0].reshape(16, -1, K_ED)
    d_cc = edge_index_c2c[1].reshape(16, -1, K_ED)
    w_cc = edge_weight_c2c.reshape(16, -1, K_ED)

    zeros = jnp.zeros((1000, HID), F32)

    xb, hbb, hbc, sb, hcb, hcc, scn = _tc1_call(
        old_data, x_centroid, p["W_atom"], p["b_atom"].reshape(1, HID),
        Wb0, bb0, Wc0, bc0)

    aggb, aggc = _get_sc_call()(
        hbb, hcb, hbc, hcc,
        s_bb, d_bb, w_bb, s_cb, d_cb, w_cb,
        s_bc, d_bc, w_bc, s_cc, d_cc, w_cc, zeros)

    xb1, hbb2, hbc2, sb2, xc1, hcb2, hcc2, sc2 = _tc2_call(
        xb, aggb[0], aggb[1], sb, x_centroid, aggc[0], aggc[1], scn,
        Wb1, bb1, Wc1, bc1)

    aggb2, aggc2 = _get_sc_call()(
        hbb2, hcb2, hbc2, hcc2,
        s_bb, d_bb, w_bb, s_cb, d_cb, w_cb,
        s_bc, d_bc, w_bc, s_cc, d_cc, w_cc, zeros)

    xbf, xcf = _tc3_call(xb1, aggb2[0], aggb2[1], sb2,
                         xc1, aggc2[0], aggc2[1], sc2)
    return (xbf, xcf)


# speedup vs baseline: 4.5383x; 1.0643x over previous
import functools

import jax
import jax.numpy as jnp
from jax import lax
from jax.experimental import pallas as pl
from jax.experimental.pallas import tpu as pltpu
from jax.experimental.pallas import tpu_sc as plsc

F32 = jnp.float32
HID = 128
NB = 10000
NC = 1000
NUM_LAYERS = 2
K_ED = 128


def _combine(Wsrc, bsrc, Wdst, bdst, Wmlp, bmlp, eps, l):
    hp = jax.lax.Precision.HIGHEST
    i_bb, i_bc, i_cc, i_cb = l * 4 + 0, l * 4 + 1, l * 4 + 2, l * 4 + 3

    def wm(i, j):
        return jnp.dot(Wsrc[i][...] if j is None else j, Wmlp[i][...],
                       preferred_element_type=F32, precision=hp)

    Mb = (1.0 + eps[0, i_bb]) * wm(i_bb, Wdst[i_bb][...]) \
       + (1.0 + eps[0, i_cb]) * wm(i_cb, Wdst[i_cb][...])
    Mc = (1.0 + eps[0, i_bc]) * wm(i_bc, Wdst[i_bc][...]) \
       + (1.0 + eps[0, i_cc]) * wm(i_cc, Wdst[i_cc][...])
    Wb = jnp.concatenate([wm(i_bb, None), wm(i_bc, None), Mb], axis=1)
    Wc = jnp.concatenate([wm(i_cb, None), wm(i_cc, None), Mc], axis=1)

    def bvec(i):
        return bsrc[i][...].reshape(1, HID)

    cb_const = wm(i_bb, bdst[i_bb][...].reshape(1, HID)) + bmlp[i_bb][...].reshape(1, HID) \
             + wm(i_cb, bdst[i_cb][...].reshape(1, HID)) + bmlp[i_cb][...].reshape(1, HID)
    cc_const = wm(i_bc, bdst[i_bc][...].reshape(1, HID)) + bmlp[i_bc][...].reshape(1, HID) \
             + wm(i_cc, bdst[i_cc][...].reshape(1, HID)) + bmlp[i_cc][...].reshape(1, HID)
    bbias = jnp.concatenate([wm(i_bb, bvec(i_bb)), wm(i_bc, bvec(i_bc)), cb_const], axis=1)
    cbias = jnp.concatenate([wm(i_cb, bvec(i_cb)), wm(i_cc, bvec(i_cc)), cc_const], axis=1)
    return Wb, bbias, Wc, cbias


def _tc1_body(old, xc, Wa, ba, Wsrc, bsrc, Wdst, bdst, Wmlp, bmlp, eps,
              xb_o, hbb_o, hbc_o, sb_o, hcb_o, hcc_o, sc_o,
              Wb1_o, bb1_o, Wc1_o, bc1_o):
    Wb0, bb0, Wc0, bc0 = _combine(Wsrc, bsrc, Wdst, bdst, Wmlp, bmlp, eps, 0)
    Wb1, bb1, Wc1, bc1 = _combine(Wsrc, bsrc, Wdst, bdst, Wmlp, bmlp, eps, 1)
    Wb1_o[...] = Wb1
    bb1_o[...] = bb1
    Wc1_o[...] = Wc1
    bc1_o[...] = bc1
    xb = jnp.dot(old[...], Wa[...], preferred_element_type=F32) + ba[...]
    xb_o[...] = xb
    yb = jnp.dot(xb, Wb0, preferred_element_type=F32) + bb0
    hbb_o[...] = yb[:, 0:HID]
    hbc_o[...] = yb[:, HID:2 * HID]
    sb_o[...] = yb[:, 2 * HID:]
    yc = jnp.dot(xc[...], Wc0, preferred_element_type=F32) + bc0
    hcb_o[...] = yc[:, 0:HID]
    hcc_o[...] = yc[:, HID:2 * HID]
    sc_o[...] = yc[:, 2 * HID:]


def _tc1_call(old, xc, Wa, ba, Wsrc, bsrc, Wdst, bdst, Wmlp, bmlp, eps):
    b = jax.ShapeDtypeStruct((NB, HID), F32)
    c = jax.ShapeDtypeStruct((NC, HID), F32)
    w = jax.ShapeDtypeStruct((HID, 3 * HID), F32)
    bi = jax.ShapeDtypeStruct((1, 3 * HID), F32)
    return pl.pallas_call(
        _tc1_body,
        out_shape=(b, b, b, b, c, c, c, w, bi, w, bi),
    )(old, xc, Wa, ba, Wsrc, bsrc, Wdst, bdst, Wmlp, bmlp, eps)


def _tc2_body(xb, ab0, ab1, sb, xc, ac0, ac1, sc, Wb, bbias, Wc, cbias,
              xb_o, hbb_o, hbc_o, sb_o, xc_o, hcb_o, hcc_o, sc_o):
    xb1 = xb[...] + jax.nn.gelu(ab0[...] + ab1[...] + sb[...])
    xb_o[...] = xb1
    yb = jnp.dot(xb1, Wb[...], preferred_element_type=F32) + bbias[...]
    hbb_o[...] = yb[:, 0:HID]
    hbc_o[...] = yb[:, HID:2 * HID]
    sb_o[...] = yb[:, 2 * HID:]
    xc1 = xc[...] + jax.nn.gelu(ac0[...] + ac1[...] + sc[...])
    xc_o[...] = xc1
    yc = jnp.dot(xc1, Wc[...], preferred_element_type=F32) + cbias[...]
    hcb_o[...] = yc[:, 0:HID]
    hcc_o[...] = yc[:, HID:2 * HID]
    sc_o[...] = yc[:, 2 * HID:]


def _tc2_call(xb, ab0, ab1, sb, xc, ac0, ac1, sc, Wb, bbias, Wc, cbias):
    b = jax.ShapeDtypeStruct((NB, HID), F32)
    c = jax.ShapeDtypeStruct((NC, HID), F32)
    return pl.pallas_call(
        _tc2_body,
        out_shape=(b, b, b, b, c, c, c, c),
    )(xb, ab0, ab1, sb, xc, ac0, ac1, sc, Wb, bbias, Wc, cbias)


def _tc3_body(xb, ab0, ab1, sb, xc, ac0, ac1, sc, xb_o, xc_o):
    xb_o[...] = xb[...] + jax.nn.gelu(ab0[...] + ab1[...] + sb[...])
    xc_o[...] = xc[...] + jax.nn.gelu(ac0[...] + ac1[...] + sc[...])


def _tc3_call(xb, ab0, ab1, sb, xc, ac0, ac1, sc):
    return pl.pallas_call(
        _tc3_body,
        out_shape=(jax.ShapeDtypeStruct((NB, HID), F32),
                   jax.ShapeDtypeStruct((NC, HID), F32)),
    )(xb, ab0, ab1, sb, xc, ac0, ac1, sc)


def _sc_body(h_bb, h_cb, h_bc, h_cc, pk_bb, pk_cb, pk_bc, pk_cc, zeros,
             aggb_o, aggc_o,
             aggb_sh, aggc_sh,
             ibuf_a, ibuf_b, rows_a, rows_b,
             isem_a, isem_b, gsem_a, gsem_b):
    c = lax.axis_index("c")
    s = lax.axis_index("s")

    @pl.when(s < 10)
    def _():
        pltpu.sync_copy(zeros.at[:], aggb_sh.at[pl.ds(s * 1000, 1000)])

    @pl.when(s < 5)
    def _():
        pltpu.sync_copy(zeros.at[pl.ds(0, 200)], aggc_sh.at[pl.ds(s * 200, 200)])

    plsc.subcore_barrier()

    def do_rel(htab, pk4, agg_sh, tiles):
        chunks = pk4.shape[1]
        wid = c * tiles + s

        def idx_start(g, ib, sm):
            pltpu.async_copy(pk4.at[wid, g], ib, sm)

        def idx_wait(ib, sm):
            pltpu.make_async_copy(pk4.at[wid, 0], ib, sm).wait()

        def gather_start(ib, rows, sm):
            pltpu.async_copy(htab.at[ib.at[0]], rows, sm)

        def gather_wait(ib, rows, sm):
            pltpu.make_async_copy(htab.at[ib.at[0]], rows, sm).wait()

        def process(ib, rows):
            def scale16(t, c2):
                wv = plsc.bitcast(ib[2, pl.ds(t * 16, 16)], F32)
                for u in range(16):
                    e = t * 16 + u
                    w = wv[u]
                    for j in range(8):
                        sl = pl.ds(j * 16, 16)
                        rows[e, sl] = rows[e, sl] * w
                return c2

            lax.fori_loop(0, K_ED // 16, scale16, 0)
            pltpu.sync_copy(rows, agg_sh.at[ib.at[1]], add=True)

        idx_start(0, ibuf_a, isem_a)
        idx_start(1, ibuf_b, isem_b)
        idx_wait(ibuf_a, isem_a)
        gather_start(ibuf_a, rows_a, gsem_a)

        def body2(t, carry):
            g = t * 2
            gather_wait(ibuf_a, rows_a, gsem_a)
            idx_wait(ibuf_b, isem_b)
            gather_start(ibuf_b, rows_b, gsem_b)
            process(ibuf_a, rows_a)

            @pl.when(g + 2 < chunks)
            def _():
                idx_start(g + 2, ibuf_a, isem_a)

            gather_wait(ibuf_b, rows_b, gsem_b)

            @pl.when(g + 2 < chunks)
            def _():
                idx_wait(ibuf_a, isem_a)
                gather_start(ibuf_a, rows_a, gsem_a)

            process(ibuf_b, rows_b)

            @pl.when(g + 3 < chunks)
            def _():
                idx_start(g + 3, ibuf_b, isem_b)

            return carry

        lax.fori_loop(0, chunks // 2, body2, 0)
        if chunks % 2:
            gather_wait(ibuf_a, rows_a, gsem_a)
            process(ibuf_a, rows_a)

    do_rel(h_bb, pk_bb, aggb_sh, 16)
    do_rel(h_cb, pk_cb, aggb_sh, 16)
    do_rel(h_bc, pk_bc, aggc_sh, 16)

    @pl.when(s < 8)
    def _():
        do_rel(h_cc, pk_cc, aggc_sh, 8)

    plsc.subcore_barrier()

    @pl.when(s < 10)
    def _():
        pltpu.sync_copy(aggb_sh.at[pl.ds(s * 1000, 1000)],
                        aggb_o.at[c, pl.ds(s * 1000, 1000)])

    @pl.when(s < 5)
    def _():
        pltpu.sync_copy(aggc_sh.at[pl.ds(s * 200, 200)],
                        aggc_o.at[c, pl.ds(s * 200, 200)])


@functools.cache
def _get_sc_call():
  return functools.partial(
    pl.kernel,
    mesh=plsc.VectorSubcoreMesh(core_axis_name="c", subcore_axis_name="s",
                                num_cores=2, num_subcores=16),
    compiler_params=pltpu.CompilerParams(use_tc_tiling_on_sc=False,
                                         needs_layout_passes=False),
    out_type=(jax.ShapeDtypeStruct((2, NB, HID), F32),
              jax.ShapeDtypeStruct((2, NC, HID), F32)),
    scratch_types=[
        pltpu.VMEM_SHARED((NB, HID), F32),
        pltpu.VMEM_SHARED((NC, HID), F32),
        pltpu.VMEM((3, K_ED), jnp.int32),
        pltpu.VMEM((3, K_ED), jnp.int32),
        pltpu.VMEM((K_ED, HID), F32),
        pltpu.VMEM((K_ED, HID), F32),
        pltpu.SemaphoreType.DMA,
        pltpu.SemaphoreType.DMA,
        pltpu.SemaphoreType.DMA,
        pltpu.SemaphoreType.DMA,
    ],
  )(_sc_body)


def _pack_edges(src, dst, w, nw):
    per = src.shape[0] // nw
    chunks = -(-per // K_ED)
    pad = chunks * K_ED - per

    def p2(x):
        return jnp.pad(x.reshape(nw, per), ((0, 0), (0, pad))).reshape(
            nw, chunks, 1, K_ED)

    wi = jax.lax.bitcast_convert_type(w, jnp.int32)
    return jnp.concatenate([p2(src), p2(dst), p2(wi)], axis=2)


def kernel(old_data, x_base, x_centroid, edge_index_b2b, edge_index_b2c,
           edge_index_c2c, edge_index_c2b, edge_weight_b2b, edge_weight_b2c,
           edge_weight_c2c, edge_weight_c2b, batch_base, batch_centroid,
           has_edge_attr, params):
    p = params
    ets = [(l, et) for l in range(NUM_LAYERS) for et in ("bb", "bc", "cc", "cb")]
    Wsrc = jnp.stack([p[f"{l}_{et}"]["W_src"] for l, et in ets])
    bsrc = jnp.stack([p[f"{l}_{et}"]["b_src"] for l, et in ets])
    Wdst = jnp.stack([p[f"{l}_{et}"]["W_dst"] for l, et in ets])
    bdst = jnp.stack([p[f"{l}_{et}"]["b_dst"] for l, et in ets])
    Wmlp = jnp.stack([p[f"{l}_{et}"]["W_mlp"] for l, et in ets])
    bmlp = jnp.stack([p[f"{l}_{et}"]["b_mlp"] for l, et in ets])
    eps = jnp.stack([p[f"{l}_{et}"]["eps"] for l, et in ets]).reshape(1, 8)

    pk_bb = _pack_edges(edge_index_b2b[0], edge_index_b2b[1], edge_weight_b2b, 32)
    pk_cb = _pack_edges(edge_index_c2b[0], edge_index_c2b[1], edge_weight_c2b, 32)
    pk_bc = _pack_edges(edge_index_b2c[0], edge_index_b2c[1], edge_weight_b2c, 32)
    pk_cc = _pack_edges(edge_index_c2c[0], edge_index_c2c[1], edge_weight_c2c, 16)

    zeros = jnp.zeros((1000, HID), F32)

    (xb, hbb, hbc, sb, hcb, hcc, scn,
     Wb1, bb1, Wc1, bc1) = _tc1_call(
        old_data, x_centroid, p["W_atom"], p["b_atom"].reshape(1, HID),
        Wsrc, bsrc, Wdst, bdst, Wmlp, bmlp, eps)

    aggb, aggc = _get_sc_call()(
        hbb, hcb, hbc, hcc, pk_bb, pk_cb, pk_bc, pk_cc, zeros)

    xb1, hbb2, hbc2, sb2, xc1, hcb2, hcc2, sc2 = _tc2_call(
        xb, aggb[0], aggb[1], sb, x_centroid, aggc[0], aggc[1], scn,
        Wb1, bb1, Wc1, bc1)

    aggb2, aggc2 = _get_sc_call()(
        hbb2, hcb2, hbc2, hcc2, pk_bb, pk_cb, pk_bc, pk_cc, zeros)

    xbf, xcf = _tc3_call(xb1, aggb2[0], aggb2[1], sb2,
                         xc1, aggc2[0], aggc2[1], sc2)
    return (xbf, xcf)
